# Initial kernel scaffold; baseline (speedup 1.0000x reference)
import jax
import jax.numpy as jnp
from jax.experimental import pallas as pl


def kernel(input, indices):
    raise NotImplementedError("write your pallas kernel here")


# speedup vs baseline: 15.1979x; 15.1979x over previous
import functools

import jax
import jax.numpy as jnp
from jax import lax
from jax.experimental import pallas as pl
from jax.experimental.pallas import tpu as pltpu
from jax.experimental.pallas import tpu_sc as plsc

N, C, D, H, W = 2, 192, 8, 28, 28
D_OUT, H_OUT, W_OUT = 16, 56, 56
PLANES = N * C
IN_PLANE = D * H * W
OUT_PLANE = D_OUT * H_OUT * W_OUT
LANES = 16
NUM_WORKERS = 32
PLANES_PER_W = PLANES // NUM_WORKERS
CHUNKS = IN_PLANE // LANES
ZCHUNKS = OUT_PLANE // LANES


def _sc_unpool_body(vals_hbm, idx_hbm, out_hbm, plane_v, idx_v, val_v):
    cid = lax.axis_index("c")
    sid = lax.axis_index("s")
    wid = sid * 2 + cid
    base = wid * PLANES_PER_W

    zero16 = jnp.zeros((LANES,), jnp.float32)

    def zero_all(i, carry):
        plane_v[pl.ds(i * LANES, LANES)] = zero16
        return carry

    lax.fori_loop(0, ZCHUNKS, zero_all, 0)

    def per_plane(j, carry):
        p = base + j
        pltpu.sync_copy(idx_hbm.at[p], idx_v)
        pltpu.sync_copy(vals_hbm.at[p], val_v)

        def scatter_vals(i, c):
            iv = idx_v[pl.ds(i * LANES, LANES)]
            vv = val_v[pl.ds(i * LANES, LANES)]
            plsc.store_scatter(plane_v, [iv], vv)
            return c

        lax.fori_loop(0, CHUNKS, scatter_vals, 0)

        pltpu.sync_copy(plane_v, out_hbm.at[p])

        def scatter_zeros(i, c):
            iv = idx_v[pl.ds(i * LANES, LANES)]
            plsc.store_scatter(plane_v, [iv], zero16)
            return c

        lax.fori_loop(0, CHUNKS, scatter_zeros, 0)
        return carry

    lax.fori_loop(0, PLANES_PER_W, per_plane, 0)


_sc_unpool = functools.partial(
    pl.kernel,
    out_type=jax.ShapeDtypeStruct((PLANES, OUT_PLANE), jnp.float32),
    mesh=plsc.VectorSubcoreMesh(core_axis_name="c", subcore_axis_name="s"),
    compiler_params=pltpu.CompilerParams(needs_layout_passes=False),
    scratch_types=[
        pltpu.VMEM((OUT_PLANE,), jnp.float32),
        pltpu.VMEM((IN_PLANE,), jnp.int32),
        pltpu.VMEM((IN_PLANE,), jnp.float32),
    ],
)(_sc_unpool_body)


def kernel(input, indices):
    vals = input.reshape(PLANES, IN_PLANE)
    idx = indices.reshape(PLANES, IN_PLANE).astype(jnp.int32)
    out = _sc_unpool(vals, idx)
    return out.reshape(N, C, D_OUT, H_OUT, W_OUT)


# speedup vs baseline: 16.8259x; 1.1071x over previous
import functools

import jax
import jax.numpy as jnp
from jax import lax
from jax.experimental import pallas as pl
from jax.experimental.pallas import tpu as pltpu
from jax.experimental.pallas import tpu_sc as plsc

N, C, D, H, W = 2, 192, 8, 28, 28
D_OUT, H_OUT, W_OUT = 16, 56, 56
PLANES = N * C
IN_PLANE = D * H * W
OUT_PLANE = D_OUT * H_OUT * W_OUT
LANES = 16
NUM_WORKERS = 32
PLANES_PER_W = PLANES // NUM_WORKERS


def _sc_unpool_body(vals_hbm, idx_hbm, out_hbm,
                    plane0, plane1, idx0, idx1, val0, val1,
                    sem_i0, sem_i1, sem_v0, sem_v1, sem_o0, sem_o1):
    cid = lax.axis_index("c")
    sid = lax.axis_index("s")
    wid = sid * 2 + cid
    base = wid * PLANES_PER_W

    planes = (plane0, plane1)
    idxs = (idx0, idx1)
    vals = (val0, val1)
    isems = (sem_i0, sem_i1)
    vsems = (sem_v0, sem_v1)
    osems = (sem_o0, sem_o1)

    zero16 = jnp.zeros((LANES,), jnp.float32)

    def issue_in(j):
        b = j & 1
        return (pltpu.async_copy(idx_hbm.at[base + j], idxs[b], isems[b]),
                pltpu.async_copy(vals_hbm.at[base + j], vals[b], vsems[b]))

    in_pending = [issue_in(0), None]
    for pv in planes:
        @plsc.parallel_loop(0, OUT_PLANE, LANES, unroll=8)
        def _(i, pv=pv):
            pv[pl.ds(i, LANES)] = zero16

    out_pending = [None, None]
    for j in range(PLANES_PER_W):
        b = j & 1
        nb = 1 - b
        pv, iv_ref, vv_ref = planes[b], idxs[b], vals[b]

        for d in in_pending[b]:
            d.wait()

        @plsc.parallel_loop(0, IN_PLANE, LANES, unroll=8)
        def _(i, pv=pv, iv_ref=iv_ref, vv_ref=vv_ref):
            iv = iv_ref[pl.ds(i, LANES)]
            vv = vv_ref[pl.ds(i, LANES)]
            plsc.store_scatter(pv, [iv], vv)

        out_pending[b] = pltpu.async_copy(pv, out_hbm.at[base + j], osems[b])

        if j >= 1:
            out_pending[nb].wait()
            out_pending[nb] = None
            if j + 1 < PLANES_PER_W:
                npv, niv_ref = planes[nb], idxs[nb]

                @plsc.parallel_loop(0, IN_PLANE, LANES, unroll=8)
                def _(i, npv=npv, niv_ref=niv_ref):
                    iv = niv_ref[pl.ds(i, LANES)]
                    plsc.store_scatter(npv, [iv], zero16)

                in_pending[nb] = issue_in(j + 1)
        elif PLANES_PER_W > 1:
            in_pending[1] = issue_in(1)

    out_pending[(PLANES_PER_W - 1) & 1].wait()


_sc_unpool = functools.partial(
    pl.kernel,
    out_type=jax.ShapeDtypeStruct((PLANES, OUT_PLANE), jnp.float32),
    mesh=plsc.VectorSubcoreMesh(core_axis_name="c", subcore_axis_name="s"),
    compiler_params=pltpu.CompilerParams(needs_layout_passes=False),
    scratch_types=[
        pltpu.VMEM((OUT_PLANE,), jnp.float32),
        pltpu.VMEM((OUT_PLANE,), jnp.float32),
        pltpu.VMEM((IN_PLANE,), jnp.int32),
        pltpu.VMEM((IN_PLANE,), jnp.int32),
        pltpu.VMEM((IN_PLANE,), jnp.float32),
        pltpu.VMEM((IN_PLANE,), jnp.float32),
        pltpu.SemaphoreType.DMA,
        pltpu.SemaphoreType.DMA,
        pltpu.SemaphoreType.DMA,
        pltpu.SemaphoreType.DMA,
        pltpu.SemaphoreType.DMA,
        pltpu.SemaphoreType.DMA,
    ],
)(_sc_unpool_body)


def kernel(input, indices):
    vals = input.reshape(PLANES, IN_PLANE)
    idx = indices.reshape(PLANES, IN_PLANE).astype(jnp.int32)
    out = _sc_unpool(vals, idx)
    return out.reshape(N, C, D_OUT, H_OUT, W_OUT)


# speedup vs baseline: 32.1660x; 1.9117x over previous
import functools

import jax
import jax.numpy as jnp
from jax import lax
from jax.experimental import pallas as pl
from jax.experimental.pallas import tpu as pltpu
from jax.experimental.pallas import tpu_sc as plsc

N, C, D, H, W = 2, 192, 8, 28, 28
D_OUT, H_OUT, W_OUT = 16, 56, 56
PLANES = N * C
IN_PLANE = D * H * W
OUT_ROWS = D_OUT * H_OUT
LANES = 16
NUM_WORKERS = 32
PLANES_PER_W = PLANES // NUM_WORKERS
SLAB_ROWS = 2 * H_OUT
SLAB_IN = H * W
TOTAL_SLABS = PLANES_PER_W * D
GROUPS = TOTAL_SLABS // 2


def _sc_unpool_body(vals_hbm, idx_hbm, out_hbm5,
                    slab0, slab1, idx_v, val_v, sem_o0, sem_o1):
    out_hbm = out_hbm5.reshape(PLANES * OUT_ROWS, W_OUT)
    cid = lax.axis_index("c")
    sid = lax.axis_index("s")
    wid = sid * 2 + cid
    base = wid * PLANES_PER_W
    rowbase = base * OUT_ROWS

    slabs = (slab0, slab1)
    osems = (sem_o0, sem_o1)

    zero16 = jnp.zeros((LANES,), jnp.float32)
    iota16 = lax.broadcasted_iota(jnp.int32, (LANES,), 0)
    inv56 = jnp.float32(1.0 / 56.0)

    for sb in slabs:
        @plsc.parallel_loop(0, SLAB_ROWS, 1, unroll=4)
        def _(r, sb=sb):
            rvec = iota16 * 0 + r
            plsc.store_scatter(sb, [rvec, iota16], zero16)
            plsc.store_scatter(sb, [rvec, iota16 + 16], zero16)
            plsc.store_scatter(sb, [rvec, iota16 + 32], zero16)
            plsc.store_scatter(sb, [rvec, iota16 + 40], zero16)

    def out_slice(t):
        return out_hbm.at[pl.ds(rowbase + t * SLAB_ROWS, SLAB_ROWS)]

    def scatter_slab(t, b):
        sb = slabs[b]
        s = t & 7
        ebase = s * SLAB_IN
        d0 = s * SLAB_ROWS

        @plsc.parallel_loop(0, SLAB_IN, LANES, unroll=7)
        def _(i, sb=sb):
            iv = idx_v[pl.ds(ebase + i, LANES)]
            vv = val_v[pl.ds(ebase + i, LANES)]
            row = (iv.astype(jnp.float32) * inv56).astype(jnp.int32)
            col = iv - row * 56
            over = (col >= 56).astype(jnp.int32)
            row_rel = row + over - d0
            col = col - over * 56
            idx_v[pl.ds(ebase + i, LANES)] = (row_rel << 6) | col
            plsc.store_scatter(sb, [row_rel, col], vv)

        pltpu.async_copy(sb, out_slice(t), osems[b])

    def drain_zero(t_prev, b):
        sb = slabs[b]
        pltpu.make_async_copy(sb, out_slice(t_prev), osems[b]).wait()
        ebase = (t_prev & 7) * SLAB_IN

        @plsc.parallel_loop(0, SLAB_IN, LANES, unroll=7)
        def _(i, sb=sb):
            packed = idx_v[pl.ds(ebase + i, LANES)]
            plsc.store_scatter(sb, [packed >> 6, packed & 63], zero16)

    @pl.loop(0, GROUPS)
    def _(g):
        t0 = g * 2

        @pl.when(g > 0)
        def _():
            drain_zero(t0 - 2, 0)
            drain_zero(t0 - 1, 1)

        @pl.when((t0 & 7) == 0)
        def _():
            j = lax.shift_right_logical(t0, 3)
            pltpu.sync_copy(idx_hbm.at[base + j], idx_v)
            pltpu.sync_copy(vals_hbm.at[base + j], val_v)

        scatter_slab(t0, 0)
        scatter_slab(t0 + 1, 1)

    pltpu.make_async_copy(slab0, out_slice(TOTAL_SLABS - 2), sem_o0).wait()
    pltpu.make_async_copy(slab1, out_slice(TOTAL_SLABS - 1), sem_o1).wait()


_sc_unpool = functools.partial(
    pl.kernel,
    out_type=jax.ShapeDtypeStruct((N, C, D_OUT, H_OUT, W_OUT), jnp.float32),
    mesh=plsc.VectorSubcoreMesh(core_axis_name="c", subcore_axis_name="s"),
    compiler_params=pltpu.CompilerParams(needs_layout_passes=False),
    scratch_types=[
        pltpu.VMEM((SLAB_ROWS, W_OUT), jnp.float32),
        pltpu.VMEM((SLAB_ROWS, W_OUT), jnp.float32),
        pltpu.VMEM((IN_PLANE,), jnp.int32),
        pltpu.VMEM((IN_PLANE,), jnp.float32),
        pltpu.SemaphoreType.DMA,
        pltpu.SemaphoreType.DMA,
    ],
)(_sc_unpool_body)


def kernel(input, indices):
    vals = input.reshape(PLANES, IN_PLANE)
    idx = indices.reshape(PLANES, IN_PLANE).astype(jnp.int32)
    return _sc_unpool(vals, idx)


# speedup vs baseline: 33.5531x; 1.0431x over previous
import jax
import jax.numpy as jnp
from jax import lax
from jax.experimental import pallas as pl
from jax.experimental.pallas import tpu as pltpu
from jax.experimental.pallas import tpu_sc as plsc

N, C, D, H, W = 2, 192, 8, 28, 28
D_OUT, H_OUT, W_OUT = 16, 56, 56
PLANES = N * C
IN_PLANE = D * H * W
OUT_ROWS = D_OUT * H_OUT
LANES = 16
NUM_WORKERS = 32
PLANES_PER_W = PLANES // NUM_WORKERS
SLAB_ROWS = 2 * H_OUT
SLAB_IN = H * W
TOTAL_SLABS = PLANES_PER_W * D


def _sc_unpool_body(vals_hbm, idx_hbm, out_hbm5,
                    slab0, slab1, idx0, idx1, val0, val1,
                    sem_o0, sem_o1, sem_i0, sem_i1, sem_v0, sem_v1):
    out_hbm = out_hbm5.reshape(PLANES * OUT_ROWS, W_OUT)
    cid = lax.axis_index("c")
    sid = lax.axis_index("s")
    wid = sid * 2 + cid
    base = wid * PLANES_PER_W
    rowbase = base * OUT_ROWS

    slabs = (slab0, slab1)
    osems = (sem_o0, sem_o1)
    idxs = (idx0, idx1)
    vals = (val0, val1)
    isems = (sem_i0, sem_i1)
    vsems = (sem_v0, sem_v1)

    zero16 = jnp.zeros((LANES,), jnp.float32)
    iota16 = lax.broadcasted_iota(jnp.int32, (LANES,), 0)
    inv56 = jnp.float32(1.0 / 56.0)

    def issue_in(j):
        b = j & 1
        return (pltpu.async_copy(idx_hbm.at[base + j], idxs[b], isems[b]),
                pltpu.async_copy(vals_hbm.at[base + j], vals[b], vsems[b]))

    pending_in = issue_in(0)
    for sb in slabs:
        @plsc.parallel_loop(0, SLAB_ROWS, 1, unroll=4)
        def _(r, sb=sb):
            rvec = iota16 * 0 + r
            plsc.store_scatter(sb, [rvec, iota16], zero16)
            plsc.store_scatter(sb, [rvec, iota16 + 16], zero16)
            plsc.store_scatter(sb, [rvec, iota16 + 32], zero16)
            plsc.store_scatter(sb, [rvec, iota16 + 40], zero16)

    def out_slice(t):
        return out_hbm.at[pl.ds(rowbase + t * SLAB_ROWS, SLAB_ROWS)]

    def scatter_slab(t, b, iv_ref, vv_ref):
        sb = slabs[b]
        d = t & 7

        @plsc.parallel_loop(0, SLAB_IN, LANES, unroll=7)
        def _(i, sb=sb, iv_ref=iv_ref, vv_ref=vv_ref, d=d):
            e = d * SLAB_IN + i
            iv = iv_ref[pl.ds(e, LANES)]
            vv = vv_ref[pl.ds(e, LANES)]
            row = (iv.astype(jnp.float32) * inv56).astype(jnp.int32)
            col = iv - row * 56
            over = (col >= 56).astype(jnp.int32)
            row = row + over - d * SLAB_ROWS
            col = col - over * 56
            iv_ref[pl.ds(e, LANES)] = (row << 6) | col
            plsc.store_scatter(sb, [row, col], vv)

        pltpu.async_copy(sb, out_slice(t), osems[b])

    def drain(t_prev, b):
        pltpu.make_async_copy(slabs[b], out_slice(t_prev), osems[b]).wait()

    def zero_slab(t_prev, b, iv_ref):
        sb = slabs[b]
        d = t_prev & 7

        @plsc.parallel_loop(0, SLAB_IN, LANES, unroll=7)
        def _(i, sb=sb, iv_ref=iv_ref, d=d):
            packed = iv_ref[pl.ds(d * SLAB_IN + i, LANES)]
            plsc.store_scatter(sb, [packed >> 6, packed & 63], zero16)

    for j in range(PLANES_PER_W):
        jb = j & 1
        iv_ref, vv_ref = idxs[jb], vals[jb]
        pv_ref = idxs[1 - jb]

        if j > 0:
            t_tail = j * 8 - 2
            drain(t_tail, 0)
            drain(t_tail + 1, 1)
            zero_slab(t_tail, 0, pv_ref)
            zero_slab(t_tail + 1, 1, pv_ref)
        if j + 1 < PLANES_PER_W:
            next_in = issue_in(j + 1)

        for dsc in pending_in:
            dsc.wait()
        if j + 1 < PLANES_PER_W:
            pending_in = next_in

        @pl.loop(0, 4)
        def _(g, iv_ref=iv_ref, vv_ref=vv_ref, j=j):
            t0 = j * 8 + g * 2

            @pl.when(g > 0)
            def _():
                drain(t0 - 2, 0)
                drain(t0 - 1, 1)
                zero_slab(t0 - 2, 0, iv_ref)
                zero_slab(t0 - 1, 1, iv_ref)

            scatter_slab(t0, 0, iv_ref, vv_ref)
            scatter_slab(t0 + 1, 1, iv_ref, vv_ref)

    drain(TOTAL_SLABS - 2, 0)
    drain(TOTAL_SLABS - 1, 1)


def _make_sc_unpool():
    return pl.kernel(
        _sc_unpool_body,
        out_type=jax.ShapeDtypeStruct((N, C, D_OUT, H_OUT, W_OUT),
                                      jnp.float32),
        mesh=plsc.VectorSubcoreMesh(core_axis_name="c", subcore_axis_name="s"),
        compiler_params=pltpu.CompilerParams(needs_layout_passes=False),
        scratch_types=[
            pltpu.VMEM((SLAB_ROWS, W_OUT), jnp.float32),
            pltpu.VMEM((SLAB_ROWS, W_OUT), jnp.float32),
            pltpu.VMEM((IN_PLANE,), jnp.int32),
            pltpu.VMEM((IN_PLANE,), jnp.int32),
            pltpu.VMEM((IN_PLANE,), jnp.float32),
            pltpu.VMEM((IN_PLANE,), jnp.float32),
            pltpu.SemaphoreType.DMA,
            pltpu.SemaphoreType.DMA,
            pltpu.SemaphoreType.DMA,
            pltpu.SemaphoreType.DMA,
            pltpu.SemaphoreType.DMA,
            pltpu.SemaphoreType.DMA,
        ],
    )


_sc_unpool = _make_sc_unpool()


def kernel(input, indices):
    vals = input.reshape(PLANES, IN_PLANE)
    idx = indices.reshape(PLANES, IN_PLANE).astype(jnp.int32)
    return _sc_unpool(vals, idx)


# speedup vs baseline: 89.7737x; 2.6756x over previous
import jax
import jax.numpy as jnp
from jax import lax
from jax.experimental import pallas as pl
from jax.experimental.pallas import tpu as pltpu
from jax.experimental.pallas import tpu_sc as plsc

N, C, D, H, W = 2, 192, 8, 28, 28
D_OUT, H_OUT, W_OUT = 16, 56, 56
LANES = 16
NUM_WORKERS = 32
ROWS = N * D * H
ROW_ELEMS = W * C
UNITS_PER_W = ROWS // NUM_WORKERS
GROUPS = UNITS_PER_W // 2
SLAB_ROWS = 4 * W_OUT
CCHUNKS = C // LANES
HW2 = H_OUT * W_OUT
OUT_POS = N * D_OUT * H_OUT * W_OUT


def _sc_unpool_body(vals_hbm, idx_hbm, out_hbm5,
                    slab0, slab1, idx0, idx1, val0,
                    sem_o0, sem_o1, sem_i, sem_v):
    out_hbm = out_hbm5.reshape(OUT_POS, C)
    cid = lax.axis_index("c")
    sid = lax.axis_index("s")
    wid = sid * 2 + cid

    slabs = (slab0, slab1)
    idxs = (idx0, idx1)
    osems = (sem_o0, sem_o1)

    zero16 = jnp.zeros((LANES,), jnp.float32)
    iota16 = lax.broadcasted_iota(jnp.int32, (LANES,), 0)
    cvecs = [iota16 + k * LANES for k in range(CCHUNKS)]

    def decode(u):
        r = u * NUM_WORKERS + wid
        n = (r >= ROWS // 2).astype(jnp.int32)
        rr = r - n * (ROWS // 2)
        d = (rr * 586) >> 14
        h = rr - d * H
        return r, n, d, h

    def out_runs(u, s):
        r, n, d, h = decode(u)
        runs = []
        for a in (0, 1):
            for b in (0, 1):
                start = ((n * D_OUT + 2 * d + a) * H_OUT + 2 * h + b) * W_OUT
                runs.append(pltpu.make_async_copy(
                    slabs[s].at[pl.ds((2 * a + b) * W_OUT, W_OUT)],
                    out_hbm.at[pl.ds(start, W_OUT)],
                    osems[s]))
        return runs

    for sb in slabs:
        @plsc.parallel_loop(0, SLAB_ROWS, 1, unroll=2)
        def _(rw, sb=sb):
            rvec = iota16 * 0 + rw
            for k in range(CCHUNKS):
                plsc.store_scatter(sb, [rvec, cvecs[k]], zero16)

    def unit_step(g, s):
        u = g * 2 + s
        sb = slabs[s]
        iv_ref = idxs[s]

        @pl.when(g > 0)
        def _():
            for dsc in out_runs(u - 2, s):
                dsc.wait()

            @plsc.parallel_loop(0, ROW_ELEMS, LANES, unroll=8)
            def _(i, sb=sb, iv_ref=iv_ref):
                packed = iv_ref[pl.ds(i, LANES)]
                plsc.store_scatter(sb, [packed >> 8, packed & 255], zero16)

        r, n, d, h = decode(u)
        pltpu.sync_copy(idx_hbm.at[r], iv_ref)
        pltpu.sync_copy(vals_hbm.at[r], val0)

        base0 = (112 * d + 2 * h) * W_OUT

        @plsc.parallel_loop(0, W, 1, unroll=2)
        def _(w, sb=sb, iv_ref=iv_ref, base0=base0):
            e0 = w * C
            basev = iota16 * 0 + (base0 + 2 * w)
            for k in range(CCHUNKS):
                iv = iv_ref[pl.ds(e0 + k * LANES, LANES)]
                vv = val0[pl.ds(e0 + k * LANES, LANES)]
                off = iv - basev
                a = (off >= HW2).astype(jnp.int32)
                off2 = off - a * HW2
                bb = (off2 >= W_OUT).astype(jnp.int32)
                cw = off2 - bb * W_OUT
                row = a * 112 + bb * 56 + (2 * w) + cw
                iv_ref[pl.ds(e0 + k * LANES, LANES)] = (row << 8) | cvecs[k]
                plsc.store_scatter(sb, [row, cvecs[k]], vv)

        for dsc in out_runs(u, s):
            dsc.start()

    @pl.loop(0, GROUPS)
    def _(g):
        unit_step(g, 0)
        unit_step(g, 1)

    for s in (0, 1):
        for dsc in out_runs(UNITS_PER_W - 2 + s, s):
            dsc.wait()


def _make_sc_unpool():
    return pl.kernel(
        _sc_unpool_body,
        out_type=jax.ShapeDtypeStruct((N, D_OUT, H_OUT, W_OUT, C),
                                      jnp.float32),
        mesh=plsc.VectorSubcoreMesh(core_axis_name="c", subcore_axis_name="s"),
        compiler_params=pltpu.CompilerParams(needs_layout_passes=False),
        scratch_types=[
            pltpu.VMEM((SLAB_ROWS, C), jnp.float32),
            pltpu.VMEM((SLAB_ROWS, C), jnp.float32),
            pltpu.VMEM((ROW_ELEMS,), jnp.int32),
            pltpu.VMEM((ROW_ELEMS,), jnp.int32),
            pltpu.VMEM((ROW_ELEMS,), jnp.float32),
            pltpu.SemaphoreType.DMA,
            pltpu.SemaphoreType.DMA,
            pltpu.SemaphoreType.DMA,
            pltpu.SemaphoreType.DMA,
        ],
    )


_sc_unpool = _make_sc_unpool()


def kernel(input, indices):
    vals = input.transpose(0, 2, 3, 4, 1).reshape(ROWS, ROW_ELEMS)
    idx = (indices.astype(jnp.int32)
           .transpose(0, 2, 3, 4, 1).reshape(ROWS, ROW_ELEMS))
    out = _sc_unpool(vals, idx)
    return out.transpose(0, 4, 1, 2, 3)


# speedup vs baseline: 99.4546x; 1.1078x over previous
import jax
import jax.numpy as jnp
from jax import lax
from jax.experimental import pallas as pl
from jax.experimental.pallas import tpu as pltpu
from jax.experimental.pallas import tpu_sc as plsc

N, C, D, H, W = 2, 192, 8, 28, 28
D_OUT, H_OUT, W_OUT = 16, 56, 56
LANES = 16
NUM_WORKERS = 32
ROWS = N * D * H
ROW_ELEMS = W * C
UNITS_PER_W = ROWS // NUM_WORKERS
GROUPS = UNITS_PER_W // 2
SLAB_ROWS = 4 * W_OUT
CCHUNKS = C // LANES
HW2 = H_OUT * W_OUT
OUT_POS = N * D_OUT * H_OUT * W_OUT


def _sc_unpool_body(vals_hbm, idx_hbm, out_hbm5,
                    slab0, slab1, idx0, idx1, val0,
                    sem_o0, sem_o1, sem_i, sem_v):
    out_hbm = out_hbm5.reshape(OUT_POS, C)
    cid = lax.axis_index("c")
    sid = lax.axis_index("s")
    wid = sid * 2 + cid

    slabs = (slab0, slab1)
    idxs = (idx0, idx1)
    osems = (sem_o0, sem_o1)

    zero16 = jnp.zeros((LANES,), jnp.float32)
    iota16 = lax.broadcasted_iota(jnp.int32, (LANES,), 0)
    cvecs = [iota16 + k * LANES for k in range(CCHUNKS)]

    def decode(u):
        r = u * NUM_WORKERS + wid
        n = (r >= ROWS // 2).astype(jnp.int32)
        rr = r - n * (ROWS // 2)
        d = (rr * 586) >> 14
        h = rr - d * H
        return r, n, d, h

    def out_runs(u, s):
        r, n, d, h = decode(u)
        runs = []
        for a in (0, 1):
            for b in (0, 1):
                start = ((n * D_OUT + 2 * d + a) * H_OUT + 2 * h + b) * W_OUT
                runs.append(pltpu.make_async_copy(
                    slabs[s].at[pl.ds((2 * a + b) * W_OUT, W_OUT)],
                    out_hbm.at[pl.ds(start, W_OUT)],
                    osems[s]))
        return runs

    for sb in slabs:
        @plsc.parallel_loop(0, SLAB_ROWS, 1, unroll=2)
        def _(rw, sb=sb):
            rvec = iota16 * 0 + rw
            for k in range(CCHUNKS):
                plsc.store_scatter(sb, [rvec, cvecs[k]], zero16)

    def unit_step(g, s):
        u = g * 2 + s
        sb = slabs[s]
        iv_ref = idxs[s]

        @pl.when(g > 0)
        def _():
            for dsc in out_runs(u - 2, s):
                dsc.wait()

            @plsc.parallel_loop(0, ROW_ELEMS, LANES, unroll=8)
            def _(i, sb=sb, iv_ref=iv_ref):
                packed = iv_ref[pl.ds(i, LANES)]
                plsc.store_scatter(sb, [packed >> 8, packed & 255], zero16)

        r, n, d, h = decode(u)
        di = pltpu.async_copy(idx_hbm.at[r], iv_ref, sem_i)
        dv = pltpu.async_copy(vals_hbm.at[r], val0, sem_v)
        di.wait()
        dv.wait()

        base0 = (112 * d + 2 * h) * W_OUT

        @plsc.parallel_loop(0, W, 1, unroll=4)
        def _(w, sb=sb, iv_ref=iv_ref, base0=base0):
            e0 = w * C
            basev = iota16 * 0 + (base0 + 2 * w)
            for k in range(CCHUNKS):
                iv = iv_ref[pl.ds(e0 + k * LANES, LANES)]
                vv = val0[pl.ds(e0 + k * LANES, LANES)]
                off = iv - basev
                a = (off >= HW2).astype(jnp.int32)
                off2 = off - a * HW2
                bb = (off2 >= W_OUT).astype(jnp.int32)
                cw = off2 - bb * W_OUT
                row = a * 112 + bb * 56 + (2 * w) + cw
                iv_ref[pl.ds(e0 + k * LANES, LANES)] = (row << 8) | cvecs[k]
                plsc.store_scatter(sb, [row, cvecs[k]], vv)

        for dsc in out_runs(u, s):
            dsc.start()

    @pl.loop(0, GROUPS)
    def _(g):
        unit_step(g, 0)
        unit_step(g, 1)

    for s in (0, 1):
        for dsc in out_runs(UNITS_PER_W - 2 + s, s):
            dsc.wait()


def _make_sc_unpool():
    return pl.kernel(
        _sc_unpool_body,
        out_type=jax.ShapeDtypeStruct((N, D_OUT, H_OUT, W_OUT, C),
                                      jnp.float32),
        mesh=plsc.VectorSubcoreMesh(core_axis_name="c", subcore_axis_name="s"),
        compiler_params=pltpu.CompilerParams(needs_layout_passes=False),
        scratch_types=[
            pltpu.VMEM((SLAB_ROWS, C), jnp.float32),
            pltpu.VMEM((SLAB_ROWS, C), jnp.float32),
            pltpu.VMEM((ROW_ELEMS,), jnp.int32),
            pltpu.VMEM((ROW_ELEMS,), jnp.int32),
            pltpu.VMEM((ROW_ELEMS,), jnp.float32),
            pltpu.SemaphoreType.DMA,
            pltpu.SemaphoreType.DMA,
            pltpu.SemaphoreType.DMA,
            pltpu.SemaphoreType.DMA,
        ],
    )


_sc_unpool = _make_sc_unpool()


def kernel(input, indices):
    vals = input.transpose(0, 2, 3, 4, 1).reshape(ROWS, ROW_ELEMS)
    idx = (indices.astype(jnp.int32)
           .transpose(0, 2, 3, 4, 1).reshape(ROWS, ROW_ELEMS))
    out = _sc_unpool(vals, idx)
    return out.transpose(0, 4, 1, 2, 3)


# speedup vs baseline: 100.1899x; 1.0074x over previous
import jax
import jax.numpy as jnp
from jax import lax
from jax.experimental import pallas as pl
from jax.experimental.pallas import tpu as pltpu
from jax.experimental.pallas import tpu_sc as plsc

N, C, D, H, W = 2, 192, 8, 28, 28
D_OUT, H_OUT, W_OUT = 16, 56, 56
LANES = 16
NUM_WORKERS = 32
ROWS = N * D * H
ROW_ELEMS = W * C
UNITS_PER_W = ROWS // NUM_WORKERS
GROUPS = UNITS_PER_W // 2
SLAB_ROWS = 4 * W_OUT
CCHUNKS = C // LANES
HW2 = H_OUT * W_OUT
OUT_POS = N * D_OUT * H_OUT * W_OUT


def _sc_unpool_body(vals_hbm, idx_hbm, out_hbm5,
                    slab0, slab1, idx0, idx1, val0,
                    sem_o0, sem_o1, sem_i, sem_v):
    out_hbm = out_hbm5.reshape(OUT_POS, C)
    cid = lax.axis_index("c")
    sid = lax.axis_index("s")
    wid = sid * 2 + cid

    slabs = (slab0, slab1)
    idxs = (idx0, idx1)
    osems = (sem_o0, sem_o1)

    zero16 = jnp.zeros((LANES,), jnp.float32)
    iota16 = lax.broadcasted_iota(jnp.int32, (LANES,), 0)
    cvecs = [iota16 + k * LANES for k in range(CCHUNKS)]

    def decode(u):
        r = u * NUM_WORKERS + wid
        n = (r >= ROWS // 2).astype(jnp.int32)
        rr = r - n * (ROWS // 2)
        d = (rr * 586) >> 14
        h = rr - d * H
        return r, n, d, h

    def out_runs(u, s):
        r, n, d, h = decode(u)
        runs = []
        for a in (0, 1):
            for b in (0, 1):
                start = ((n * D_OUT + 2 * d + a) * H_OUT + 2 * h + b) * W_OUT
                runs.append(pltpu.make_async_copy(
                    slabs[s].at[pl.ds((2 * a + b) * W_OUT, W_OUT)],
                    out_hbm.at[pl.ds(start, W_OUT)],
                    osems[s]))
        return runs

    for sb in slabs:
        @plsc.parallel_loop(0, SLAB_ROWS, 1, unroll=2)
        def _(rw, sb=sb):
            rvec = iota16 * 0 + rw
            for k in range(CCHUNKS):
                plsc.store_scatter(sb, [rvec, cvecs[k]], zero16)

    def unit_step(g, s):
        u = g * 2 + s
        sb = slabs[s]
        iv_ref = idxs[s]

        r, n, d, h = decode(u)
        dv = pltpu.async_copy(vals_hbm.at[r], val0, sem_v)

        @pl.when(g > 0)
        def _():
            for dsc in out_runs(u - 2, s):
                dsc.wait()

            @plsc.parallel_loop(0, ROW_ELEMS, LANES, unroll=8)
            def _(i, sb=sb, iv_ref=iv_ref):
                packed = iv_ref[pl.ds(i, LANES)]
                plsc.store_scatter(sb, [packed >> 8, packed & 255], zero16)

        di = pltpu.async_copy(idx_hbm.at[r], iv_ref, sem_i)
        di.wait()
        dv.wait()

        base0 = (112 * d + 2 * h) * W_OUT

        @plsc.parallel_loop(0, W, 1, unroll=4)
        def _(w, sb=sb, iv_ref=iv_ref, base0=base0):
            e0 = w * C
            basev = iota16 * 0 + (base0 + 2 * w)
            for k in range(CCHUNKS):
                iv = iv_ref[pl.ds(e0 + k * LANES, LANES)]
                vv = val0[pl.ds(e0 + k * LANES, LANES)]
                off = iv - basev
                a = (off >= HW2).astype(jnp.int32)
                off2 = off - a * HW2
                bb = (off2 >= W_OUT).astype(jnp.int32)
                cw = off2 - bb * W_OUT
                row = a * 112 + bb * 56 + (2 * w) + cw
                iv_ref[pl.ds(e0 + k * LANES, LANES)] = (row << 8) | cvecs[k]
                plsc.store_scatter(sb, [row, cvecs[k]], vv)

        for dsc in out_runs(u, s):
            dsc.start()

    @pl.loop(0, GROUPS)
    def _(g):
        unit_step(g, 0)
        unit_step(g, 1)

    for s in (0, 1):
        for dsc in out_runs(UNITS_PER_W - 2 + s, s):
            dsc.wait()


def _make_sc_unpool():
    return pl.kernel(
        _sc_unpool_body,
        out_type=jax.ShapeDtypeStruct((N, D_OUT, H_OUT, W_OUT, C),
                                      jnp.float32),
        mesh=plsc.VectorSubcoreMesh(core_axis_name="c", subcore_axis_name="s"),
        compiler_params=pltpu.CompilerParams(needs_layout_passes=False),
        scratch_types=[
            pltpu.VMEM((SLAB_ROWS, C), jnp.float32),
            pltpu.VMEM((SLAB_ROWS, C), jnp.float32),
            pltpu.VMEM((ROW_ELEMS,), jnp.int32),
            pltpu.VMEM((ROW_ELEMS,), jnp.int32),
            pltpu.VMEM((ROW_ELEMS,), jnp.float32),
            pltpu.SemaphoreType.DMA,
            pltpu.SemaphoreType.DMA,
            pltpu.SemaphoreType.DMA,
            pltpu.SemaphoreType.DMA,
        ],
    )


_sc_unpool = _make_sc_unpool()


def kernel(input, indices):
    vals = input.transpose(0, 2, 3, 4, 1).reshape(ROWS, ROW_ELEMS)
    idx = (indices.astype(jnp.int32)
           .transpose(0, 2, 3, 4, 1).reshape(ROWS, ROW_ELEMS))
    out = _sc_unpool(vals, idx)
    return out.transpose(0, 4, 1, 2, 3)
